# baseline (speedup 1.0000x reference)
import functools

import jax
import jax.numpy as jnp
from jax import lax
from jax.experimental import pallas as pl
from jax.experimental.pallas import tpu as pltpu
from jax.experimental.pallas import tpu_sc as plsc

_A = 128
_DM = 4096
_RK = 64
_B = 64
_NW = 32
_IDW = _B // _NW
_BANDS = _RK // 8
_CHALF = _DM // 2


_NBUF = 6
_LEAD = 3
_CCOL = 1024
_NCH = _BANDS * (_DM // _CCOL)


def _body(w_hbm, idx_hbm, out_hbm, *args):
    idx_v = args[0]
    bufs = args[1:1 + _NBUF]
    gsems = args[1 + _NBUF:1 + 2 * _NBUF]
    wsems = args[1 + 2 * _NBUF:1 + 3 * _NBUF]
    wid = lax.axis_index("s") * 2 + lax.axis_index("c")
    ncol = _DM // _CCOL

    pltpu.sync_copy(idx_hbm.at[wid], idx_v)

    def src(c):
        band, h = c // ncol, c % ncol
        return w_hbm.at[idx_v,
                        pl.ds(band * 8, 8),
                        pl.ds(h * _CCOL, _CCOL)]

    def dst(c):
        band, h = c // ncol, c % ncol
        return out_hbm.at[pl.ds(wid * _IDW, _IDW),
                          pl.ds(band * 8, 8),
                          pl.ds(h * _CCOL, _CCOL)]

    gathers = [None] * _NBUF
    writes = [None] * _NBUF

    for c in range(_LEAD):
        gathers[c % _NBUF] = pltpu.async_copy(src(c), bufs[c % _NBUF],
                                              gsems[c % _NBUF])
    for c in range(_NCH):
        s = c % _NBUF
        nxt = c + _LEAD
        if nxt < _NCH:
            sn = nxt % _NBUF
            if writes[sn] is not None:
                writes[sn].wait()
            gathers[sn] = pltpu.async_copy(src(nxt), bufs[sn], gsems[sn])
        gathers[s].wait()
        writes[s] = pltpu.async_copy(bufs[s], dst(c), wsems[s])
    for s in range(_NBUF):
        if writes[s] is not None:
            writes[s].wait()


@jax.jit
def _sc_gather(wv, idx2):
    mesh = plsc.VectorSubcoreMesh(core_axis_name="c", subcore_axis_name="s")
    f = functools.partial(
        pl.kernel,
        mesh=mesh,
        out_type=jax.ShapeDtypeStruct((_B, _RK, _DM), jnp.float32),
        scratch_types=(
            [pltpu.VMEM((_IDW,), jnp.int32)]
            + [pltpu.VMEM((_IDW, 8, _CCOL), jnp.float32)] * _NBUF
            + [pltpu.SemaphoreType.DMA] * (2 * _NBUF)
        ),
    )(_body)
    return f(wv, idx2)


def kernel(weight, adapter_ids):
    wv = jnp.swapaxes(weight, 1, 2)
    idx2 = adapter_ids.astype(jnp.int32).reshape(_NW, _IDW)
    out = _sc_gather(wv, idx2)
    return jnp.swapaxes(out, 1, 2)


# speedup vs baseline: 1.0004x; 1.0004x over previous
import functools

import jax
import jax.numpy as jnp
from jax import lax
from jax.experimental import pallas as pl
from jax.experimental.pallas import tpu as pltpu
from jax.experimental.pallas import tpu_sc as plsc

_A = 128
_DM = 4096
_RK = 64
_B = 64
_NW = 32
_IDW = _B // _NW
_BANDS = _RK // 8
_CHALF = _DM // 2


_NBUF = 3
_LEAD = 2
_CCOL = 2048
_NCH = _BANDS * (_DM // _CCOL)


def _body(w_hbm, idx_hbm, out_hbm, *args):
    idx_v = args[0]
    bufs = args[1:1 + _NBUF]
    gsems = args[1 + _NBUF:1 + 2 * _NBUF]
    wsems = args[1 + 2 * _NBUF:1 + 3 * _NBUF]
    wid = lax.axis_index("s") * 2 + lax.axis_index("c")
    ncol = _DM // _CCOL

    pltpu.sync_copy(idx_hbm.at[wid], idx_v)

    def band_of(c):
        return lax.rem(c // ncol + wid, _BANDS)

    def src(c):
        return w_hbm.at[idx_v,
                        pl.ds(band_of(c) * 8, 8),
                        pl.ds((c % ncol) * _CCOL, _CCOL)]

    def dst(c):
        return out_hbm.at[pl.ds(wid * _IDW, _IDW),
                          pl.ds(band_of(c) * 8, 8),
                          pl.ds((c % ncol) * _CCOL, _CCOL)]

    gathers = [None] * _NBUF
    writes = [None] * _NBUF

    for c in range(_LEAD):
        gathers[c % _NBUF] = pltpu.async_copy(src(c), bufs[c % _NBUF],
                                              gsems[c % _NBUF])
    for c in range(_NCH):
        s = c % _NBUF
        nxt = c + _LEAD
        if nxt < _NCH:
            sn = nxt % _NBUF
            if writes[sn] is not None:
                writes[sn].wait()
            gathers[sn] = pltpu.async_copy(src(nxt), bufs[sn], gsems[sn])
        gathers[s].wait()
        writes[s] = pltpu.async_copy(bufs[s], dst(c), wsems[s])
    for s in range(_NBUF):
        if writes[s] is not None:
            writes[s].wait()


@jax.jit
def _sc_gather(wv, idx2):
    mesh = plsc.VectorSubcoreMesh(core_axis_name="c", subcore_axis_name="s")
    f = functools.partial(
        pl.kernel,
        mesh=mesh,
        out_type=jax.ShapeDtypeStruct((_B, _RK, _DM), jnp.float32),
        scratch_types=(
            [pltpu.VMEM((_IDW,), jnp.int32)]
            + [pltpu.VMEM((_IDW, 8, _CCOL), jnp.float32)] * _NBUF
            + [pltpu.SemaphoreType.DMA] * (2 * _NBUF)
        ),
    )(_body)
    return f(wv, idx2)


def kernel(weight, adapter_ids):
    wv = jnp.swapaxes(weight, 1, 2)
    idx2 = adapter_ids.astype(jnp.int32).reshape(_NW, _IDW)
    out = _sc_gather(wv, idx2)
    return jnp.swapaxes(out, 1, 2)
